# baseline (speedup 1.0000x reference)
import functools

import jax
import jax.numpy as jnp
from jax import lax
from jax.experimental import pallas as pl
from jax.experimental.pallas import tpu as pltpu
from jax.experimental.pallas import tpu_sc as plsc

N = 10000
D = 128
NC = 2
NS = 16
NW = NC * NS
K = 128

ZROWS = 632
ACC_ROWS = NS * ZROWS

DEG_ROWS = 10240
DEG_Z = DEG_ROWS // NS


def _mesh():
    return plsc.VectorSubcoreMesh(core_axis_name="c", subcore_axis_name="s")


def _make_deg_kernel(ch, e):
    @functools.partial(
        pl.kernel,
        out_type=jax.ShapeDtypeStruct((NC, DEG_ROWS), jnp.float32),
        mesh=_mesh(),
        scratch_types=[
            pltpu.VMEM((ch, K), jnp.int32),
            pltpu.VMEM((K,), jnp.float32),
            pltpu.VMEM_SHARED((DEG_ROWS,), jnp.float32),
        ],
    )
    def deg_kernel(dst_hbm, zeros_hbm, out_hbm, dst_v, ones_v, acc):
        c = lax.axis_index("c")
        s = lax.axis_index("s")
        w = s * NC + c
        n_w = jnp.minimum(ch, jnp.maximum(0, (e - w * (ch * K) + K - 1) // K))
        pltpu.sync_copy(dst_hbm.at[w], dst_v)
        for i in range(K // 16):
            ones_v[pl.ds(i * 16, 16)] = jnp.ones((16,), jnp.float32)
        pltpu.sync_copy(zeros_hbm, acc.at[pl.ds(s * DEG_Z, DEG_Z)])
        plsc.subcore_barrier()

        def body(j, carry):
            pltpu.sync_copy(ones_v, acc.at[dst_v.at[j]], add=True)
            return carry

        lax.fori_loop(0, n_w, body, 0)
        plsc.subcore_barrier()
        pltpu.sync_copy(acc.at[pl.ds(s * DEG_Z, DEG_Z)],
                        out_hbm.at[c].at[pl.ds(s * DEG_Z, DEG_Z)])

    return deg_kernel


W_CH = 16


def _make_scatter_kernel(ch, e):
    @functools.partial(
        pl.kernel,
        out_type=jax.ShapeDtypeStruct((NC, ACC_ROWS, D), jnp.float32),
        mesh=_mesh(),
        scratch_types=[
            pltpu.VMEM((W_CH, K), jnp.int32),
            pltpu.VMEM((W_CH, K), jnp.int32),
            pltpu.VMEM((W_CH, K), jnp.int32),
            pltpu.VMEM((W_CH, K), jnp.int32),
            pltpu.VMEM((K, D), jnp.float32),
            pltpu.VMEM((K, D), jnp.float32),
            pltpu.VMEM_SHARED((ACC_ROWS, D), jnp.float32),
            pltpu.SemaphoreType.DMA,
            pltpu.SemaphoreType.DMA,
            pltpu.SemaphoreType.DMA,
            pltpu.SemaphoreType.DMA,
        ],
    )
    def scatter_kernel(g_hbm, src_hbm, dst_hbm, zeros_hbm, out_hbm,
                       src0, dst0, src1, dst1, buf_a, buf_b, acc,
                       sem_a, sem_b, sem_i0, sem_i1):
        c = lax.axis_index("c")
        s = lax.axis_index("s")
        w = s * NC + c
        n_w = jnp.minimum(ch, jnp.maximum(0, (e - w * (ch * K) + K - 1) // K))
        n_win = (n_w + W_CH - 1) // W_CH

        def load_idx(wi, sv, dv, sem):
            base = wi * W_CH
            pltpu.async_copy(src_hbm.at[w].at[pl.ds(base, W_CH)], sv, sem)
            pltpu.async_copy(dst_hbm.at[w].at[pl.ds(base, W_CH)], dv, sem)

        def wait_idx(sv, dv, sem):
            pltpu.make_async_copy(src_hbm.at[w].at[pl.ds(0, W_CH)], sv,
                                  sem).wait()
            pltpu.make_async_copy(dst_hbm.at[w].at[pl.ds(0, W_CH)], dv,
                                  sem).wait()

        def process(wi, sv, dv):
            m = jnp.minimum(W_CH, n_w - wi * W_CH)
            pltpu.async_copy(g_hbm.at[pl.ds(0, K)], buf_a, sem_a)

            def pair(t, carry):
                j0 = 2 * t
                j1 = j0 + 1

                @pl.when(j1 < m)
                def _():
                    pltpu.async_copy(g_hbm.at[pl.ds(0, K)], buf_b, sem_b)

                pltpu.make_async_copy(g_hbm.at[pl.ds(0, K)], buf_a, sem_a).wait()
                pltpu.sync_copy(buf_a, acc.at[dv.at[j0]], add=True)

                @pl.when(j0 + 2 < m)
                def _():
                    pltpu.async_copy(g_hbm.at[pl.ds(0, K)], buf_a, sem_a)

                @pl.when(j1 < m)
                def _():
                    pltpu.make_async_copy(g_hbm.at[pl.ds(0, K)], buf_b,
                                          sem_b).wait()
                    pltpu.sync_copy(buf_b, acc.at[dv.at[j1]], add=True)

                return carry

            lax.fori_loop(0, (m + 1) // 2, pair, 0)

        @pl.when(n_win > 0)
        def _():
            load_idx(0, src0, dst0, sem_i0)
        pltpu.sync_copy(zeros_hbm, acc.at[pl.ds(s * ZROWS, ZROWS)])
        plsc.subcore_barrier()

        def win_pair(t, carry):
            i0 = 2 * t
            i1 = i0 + 1

            @pl.when(i1 < n_win)
            def _():
                load_idx(i1, src1, dst1, sem_i1)

            wait_idx(src0, dst0, sem_i0)
            process(i0, src0, dst0)

            @pl.when(i0 + 2 < n_win)
            def _():
                load_idx(i0 + 2, src0, dst0, sem_i0)

            @pl.when(i1 < n_win)
            def _():
                wait_idx(src1, dst1, sem_i1)
                process(i1, src1, dst1)

            return carry

        lax.fori_loop(0, (n_win + 1) // 2, win_pair, 0)
        plsc.subcore_barrier()
        pltpu.sync_copy(acc.at[pl.ds(s * ZROWS, ZROWS)],
                        out_hbm.at[c].at[pl.ds(s * ZROWS, ZROWS)])

    return scatter_kernel


BM = 2000


def _first_tc(x_ref, w_ref, degp_ref, g_ref):
    deg = degp_ref[0] + degp_ref[1] + 1.0
    d = lax.rsqrt(deg)
    g_ref[...] = (x_ref[...] @ w_ref[...]) * d


def _mid_tc(s_ref, g_ref, w_ref, b_ref, degp_ref, o_ref):
    deg = degp_ref[0] + degp_ref[1] + 1.0
    d = lax.rsqrt(deg)
    z = d * (s_ref[0] + s_ref[1] + g_ref[...]) + b_ref[...]
    h = jnp.maximum(z, 0.0) @ w_ref[...]
    o_ref[...] = h * d


def _last_tc(s_ref, g_ref, b_ref, degp_ref, o_ref):
    deg = degp_ref[0] + degp_ref[1] + 1.0
    d = lax.rsqrt(deg)
    o_ref[...] = d * (s_ref[0] + s_ref[1] + g_ref[...]) + b_ref[...]


def _row_grid():
    return N // BM


_SPEC_S = pl.BlockSpec((2, BM, D), lambda i: (0, i, 0))
_SPEC_ROWS = pl.BlockSpec((BM, D), lambda i: (i, 0))
_SPEC_W = pl.BlockSpec((D, D), lambda i: (0, 0))
_SPEC_B = pl.BlockSpec((1, D), lambda i: (0, 0))
_SPEC_DEG = pl.BlockSpec((2, BM, 1), lambda i: (0, i, 0))


def kernel(x, edge_index, W1, b1, W2, b2, W3, b3):
    src = edge_index[0]
    dst = edge_index[1]
    e = src.shape[0]
    ch = -(-e // (NW * K))
    ch = ((ch + W_CH - 1) // W_CH) * W_CH
    e_pad = NW * ch * K
    pad = e_pad - e
    pad_dst = N + (jnp.arange(pad, dtype=jnp.int32) % (ACC_ROWS - N))
    src_p = jnp.concatenate([src, jnp.zeros((pad,), jnp.int32)]).reshape(NW, ch, K)
    dst_p = jnp.concatenate([dst, pad_dst]).reshape(NW, ch, K)

    zeros_rows = jnp.zeros((ZROWS, D), jnp.float32)
    zeros_deg = jnp.zeros((DEG_Z,), jnp.float32)

    degp = _make_deg_kernel(ch, e)(dst_p, zeros_deg)
    degp = degp[:, :N, None]

    scatter = _make_scatter_kernel(ch, e)

    first = pl.pallas_call(
        _first_tc,
        grid=(_row_grid(),),
        in_specs=[_SPEC_ROWS, _SPEC_W, _SPEC_DEG],
        out_specs=_SPEC_ROWS,
        out_shape=jax.ShapeDtypeStruct((N, D), jnp.float32),
    )
    mid = pl.pallas_call(
        _mid_tc,
        grid=(_row_grid(),),
        in_specs=[_SPEC_S, _SPEC_ROWS, _SPEC_W, _SPEC_B, _SPEC_DEG],
        out_specs=_SPEC_ROWS,
        out_shape=jax.ShapeDtypeStruct((N, D), jnp.float32),
    )
    last = pl.pallas_call(
        _last_tc,
        grid=(_row_grid(),),
        in_specs=[_SPEC_S, _SPEC_ROWS, _SPEC_B, _SPEC_DEG],
        out_specs=_SPEC_ROWS,
        out_shape=jax.ShapeDtypeStruct((N, D), jnp.float32),
    )

    g1 = first(x, W1, degp)
    s1 = scatter(g1, src_p, dst_p, zeros_rows)
    g2 = mid(s1, g1, W2, b1.reshape(1, D), degp)
    s2 = scatter(g2, src_p, dst_p, zeros_rows)
    g3 = mid(s2, g2, W3, b2.reshape(1, D), degp)
    s3 = scatter(g3, src_p, dst_p, zeros_rows)
    return last(s3, g3, b3.reshape(1, D), degp)


# speedup vs baseline: 1.9157x; 1.9157x over previous
import functools

import jax
import jax.numpy as jnp
from jax import lax
from jax.experimental import pallas as pl
from jax.experimental.pallas import tpu as pltpu
from jax.experimental.pallas import tpu_sc as plsc

N = 10000
D = 128
NC = 2
NS = 16
NW = NC * NS
K = 128

ZROWS = 632
ACC_ROWS = NS * ZROWS

DEG_ROWS = 10240
DEG_Z = DEG_ROWS // NS


def _mesh():
    return plsc.VectorSubcoreMesh(core_axis_name="c", subcore_axis_name="s")


def _make_deg_kernel(ch, e):
    @functools.partial(
        pl.kernel,
        out_type=jax.ShapeDtypeStruct((NC, DEG_ROWS), jnp.float32),
        mesh=_mesh(),
        scratch_types=[
            pltpu.VMEM((ch, K), jnp.int32),
            pltpu.VMEM((K,), jnp.float32),
            pltpu.VMEM_SHARED((DEG_ROWS,), jnp.float32),
        ],
    )
    def deg_kernel(dst_hbm, zeros_hbm, out_hbm, dst_v, ones_v, acc):
        c = lax.axis_index("c")
        s = lax.axis_index("s")
        w = s * NC + c
        n_w = jnp.minimum(ch, jnp.maximum(0, (e - w * (ch * K) + K - 1) // K))
        pltpu.sync_copy(dst_hbm.at[w], dst_v)
        for i in range(K // 16):
            ones_v[pl.ds(i * 16, 16)] = jnp.ones((16,), jnp.float32)
        pltpu.sync_copy(zeros_hbm, acc.at[pl.ds(s * DEG_Z, DEG_Z)])
        plsc.subcore_barrier()

        def body(j, carry):
            pltpu.sync_copy(ones_v, acc.at[dst_v.at[j]], add=True)
            return carry

        lax.fori_loop(0, n_w, body, 0)
        plsc.subcore_barrier()
        pltpu.sync_copy(acc.at[pl.ds(s * DEG_Z, DEG_Z)],
                        out_hbm.at[c].at[pl.ds(s * DEG_Z, DEG_Z)])

    return deg_kernel


W_CH = 16


def _make_scatter_kernel(ch, e):
    @functools.partial(
        pl.kernel,
        out_type=jax.ShapeDtypeStruct((NC, ACC_ROWS, D), jnp.float32),
        mesh=_mesh(),
        scratch_types=[
            pltpu.VMEM((W_CH, K), jnp.int32),
            pltpu.VMEM((W_CH, K), jnp.int32),
            pltpu.VMEM((W_CH, K), jnp.int32),
            pltpu.VMEM((W_CH, K), jnp.int32),
            pltpu.VMEM((K, D), jnp.float32),
            pltpu.VMEM((K, D), jnp.float32),
            pltpu.VMEM((40, D), jnp.float32),
            pltpu.VMEM_SHARED((ACC_ROWS, D), jnp.float32),
            pltpu.SemaphoreType.DMA,
            pltpu.SemaphoreType.DMA,
            pltpu.SemaphoreType.DMA,
            pltpu.SemaphoreType.DMA,
        ],
    )
    def scatter_kernel(g_hbm, src_hbm, dst_hbm, out_hbm,
                       src0, dst0, src1, dst1, buf_a, buf_b, zbuf, acc,
                       sem_a, sem_b, sem_i0, sem_i1):
        c = lax.axis_index("c")
        s = lax.axis_index("s")
        w = s * NC + c
        n_w = jnp.minimum(ch, jnp.maximum(0, (e - w * (ch * K) + K - 1) // K))
        n_win = (n_w + W_CH - 1) // W_CH

        def load_idx(wi, sv, dv, sem):
            base = wi * W_CH
            pltpu.async_copy(src_hbm.at[w].at[pl.ds(base, W_CH)], sv, sem)
            pltpu.async_copy(dst_hbm.at[w].at[pl.ds(base, W_CH)], dv, sem)

        def wait_idx(sv, dv, sem):
            pltpu.make_async_copy(src_hbm.at[w].at[pl.ds(0, W_CH)], sv,
                                  sem).wait()
            pltpu.make_async_copy(dst_hbm.at[w].at[pl.ds(0, W_CH)], dv,
                                  sem).wait()

        def process(wi, sv, dv):
            m = jnp.minimum(W_CH, n_w - wi * W_CH)
            pltpu.async_copy(g_hbm.at[sv.at[0]], buf_a, sem_a)

            def pair(t, carry):
                j0 = 2 * t
                j1 = j0 + 1

                @pl.when(j1 < m)
                def _():
                    pltpu.async_copy(g_hbm.at[sv.at[j1]], buf_b, sem_b)

                pltpu.make_async_copy(g_hbm.at[sv.at[j0]], buf_a, sem_a).wait()
                pltpu.sync_copy(buf_a, acc.at[dv.at[j0]], add=True)

                @pl.when(j0 + 2 < m)
                def _():
                    pltpu.async_copy(g_hbm.at[sv.at[j0 + 2]], buf_a, sem_a)

                @pl.when(j1 < m)
                def _():
                    pltpu.make_async_copy(g_hbm.at[sv.at[j1]], buf_b,
                                          sem_b).wait()
                    pltpu.sync_copy(buf_b, acc.at[dv.at[j1]], add=True)

                return carry

            lax.fori_loop(0, (m + 1) // 2, pair, 0)

        @pl.when(n_win > 0)
        def _():
            load_idx(0, src0, dst0, sem_i0)
        for i in range(40):
            for k in range(D // 16):
                zbuf[i, pl.ds(k * 16, 16)] = jnp.zeros((16,), jnp.float32)
        for r in range(15):
            pltpu.sync_copy(zbuf, acc.at[pl.ds(s * ZROWS + r * 40, 40)])
        pltpu.sync_copy(zbuf.at[pl.ds(0, 32)],
                        acc.at[pl.ds(s * ZROWS + 600, 32)])
        plsc.subcore_barrier()

        def win_pair(t, carry):
            i0 = 2 * t
            i1 = i0 + 1

            @pl.when(i1 < n_win)
            def _():
                load_idx(i1, src1, dst1, sem_i1)

            wait_idx(src0, dst0, sem_i0)
            process(i0, src0, dst0)

            @pl.when(i0 + 2 < n_win)
            def _():
                load_idx(i0 + 2, src0, dst0, sem_i0)

            @pl.when(i1 < n_win)
            def _():
                wait_idx(src1, dst1, sem_i1)
                process(i1, src1, dst1)

            return carry

        lax.fori_loop(0, (n_win + 1) // 2, win_pair, 0)
        plsc.subcore_barrier()
        pltpu.sync_copy(acc.at[pl.ds(s * ZROWS, ZROWS)],
                        out_hbm.at[c].at[pl.ds(s * ZROWS, ZROWS)])

    return scatter_kernel


BM = 2000


def _first_tc(x_ref, w_ref, degp_ref, g_ref):
    deg = degp_ref[0] + degp_ref[1] + 1.0
    d = lax.rsqrt(deg)
    g_ref[...] = (x_ref[...] @ w_ref[...]) * d


def _mid_tc(s_ref, g_ref, w_ref, b_ref, degp_ref, o_ref):
    deg = degp_ref[0] + degp_ref[1] + 1.0
    d = lax.rsqrt(deg)
    z = d * (s_ref[0] + s_ref[1] + g_ref[...]) + b_ref[...]
    h = jnp.maximum(z, 0.0) @ w_ref[...]
    o_ref[...] = h * d


def _last_tc(s_ref, g_ref, b_ref, degp_ref, o_ref):
    deg = degp_ref[0] + degp_ref[1] + 1.0
    d = lax.rsqrt(deg)
    o_ref[...] = d * (s_ref[0] + s_ref[1] + g_ref[...]) + b_ref[...]


def _row_grid():
    return N // BM


_SPEC_S = pl.BlockSpec((2, BM, D), lambda i: (0, i, 0))
_SPEC_ROWS = pl.BlockSpec((BM, D), lambda i: (i, 0))
_SPEC_W = pl.BlockSpec((D, D), lambda i: (0, 0))
_SPEC_B = pl.BlockSpec((1, D), lambda i: (0, 0))
_SPEC_DEG = pl.BlockSpec((2, BM, 1), lambda i: (0, i, 0))


def kernel(x, edge_index, W1, b1, W2, b2, W3, b3):
    src = edge_index[0]
    dst = edge_index[1]
    e = src.shape[0]
    ch = -(-e // (NW * K))
    ch = ((ch + W_CH - 1) // W_CH) * W_CH
    e_pad = NW * ch * K
    pad = e_pad - e
    pad_dst = N + (jnp.arange(pad, dtype=jnp.int32) % (ACC_ROWS - N))
    src_p = jnp.concatenate([src, jnp.zeros((pad,), jnp.int32)]).reshape(NW, ch, K)
    dst_p = jnp.concatenate([dst, pad_dst]).reshape(NW, ch, K)

    zeros_deg = jnp.zeros((DEG_Z,), jnp.float32)

    degp = _make_deg_kernel(ch, e)(dst_p, zeros_deg)
    degp = degp[:, :N, None]

    scatter = _make_scatter_kernel(ch, e)

    first = pl.pallas_call(
        _first_tc,
        grid=(_row_grid(),),
        in_specs=[_SPEC_ROWS, _SPEC_W, _SPEC_DEG],
        out_specs=_SPEC_ROWS,
        out_shape=jax.ShapeDtypeStruct((N, D), jnp.float32),
    )
    mid = pl.pallas_call(
        _mid_tc,
        grid=(_row_grid(),),
        in_specs=[_SPEC_S, _SPEC_ROWS, _SPEC_W, _SPEC_B, _SPEC_DEG],
        out_specs=_SPEC_ROWS,
        out_shape=jax.ShapeDtypeStruct((N, D), jnp.float32),
    )
    last = pl.pallas_call(
        _last_tc,
        grid=(_row_grid(),),
        in_specs=[_SPEC_S, _SPEC_ROWS, _SPEC_B, _SPEC_DEG],
        out_specs=_SPEC_ROWS,
        out_shape=jax.ShapeDtypeStruct((N, D), jnp.float32),
    )

    g1 = first(x, W1, degp)
    s1 = scatter(g1, src_p, dst_p)
    g2 = mid(s1, g1, W2, b1.reshape(1, D), degp)
    s2 = scatter(g2, src_p, dst_p)
    g3 = mid(s2, g2, W3, b2.reshape(1, D), degp)
    s3 = scatter(g3, src_p, dst_p)
    return last(s3, g3, b3.reshape(1, D), degp)
